# baseline (speedup 1.0000x reference)
import functools

import jax
import jax.numpy as jnp
from jax import lax
from jax.experimental import pallas as pl
from jax.experimental.pallas import tpu as pltpu
from jax.experimental.pallas import tpu_sc as plsc

N_NODES = 10000
N_EDGES = 320000
D = 128

NC = 2
NS = 16
NW = NC * NS
EPT = N_EDGES // NW

CH_D = 125
NCH_D = EPT // CH_D

CH_A = 80
NCH_A = EPT // CH_A

ROWS_PT = N_NODES // NS


@functools.cache
def _sc_kernels():
    mesh = plsc.VectorSubcoreMesh(
        core_axis_name="c", subcore_axis_name="s", num_cores=NC,
        num_subcores=NS
    )

    @functools.partial(
        pl.kernel,
        out_type=jax.ShapeDtypeStruct((NC, 2, N_NODES), jnp.float32),
        mesh=mesh,
        scratch_types=[
            pltpu.VMEM((NCH_D, CH_D), jnp.int32),
            pltpu.VMEM((NCH_D, CH_D), jnp.int32),
            pltpu.VMEM((CH_D,), jnp.float32),
            pltpu.VMEM_SHARED((N_NODES,), jnp.float32),
            pltpu.VMEM_SHARED((N_NODES,), jnp.float32),
            pltpu.SemaphoreType.DMA,
            pltpu.SemaphoreType.DMA,
        ],
    )
    def deg_kernel(src_h, dst_h, z1d_h, ones_h, out_h, sidx_v, didx_v,
                   ones_v, dego_sh, degi_sh, sema, semb):
        c = lax.axis_index("c")
        s = lax.axis_index("s")
        wid = c * NS + s

        @pl.when(s == 0)
        def _zero():
            pltpu.sync_copy(z1d_h, dego_sh)
            pltpu.sync_copy(z1d_h, degi_sh)

        pltpu.sync_copy(src_h.at[wid], sidx_v)
        pltpu.sync_copy(dst_h.at[wid], didx_v)
        pltpu.sync_copy(ones_h, ones_v)
        plsc.subcore_barrier()

        def issue(i):
            pltpu.async_copy(ones_v, dego_sh.at[sidx_v.at[i]], sema,
                             add=True)
            pltpu.async_copy(ones_v, degi_sh.at[didx_v.at[i]], semb,
                             add=True)

        def drain(i):
            pltpu.make_async_copy(ones_v, dego_sh.at[sidx_v.at[i]],
                                  sema).wait()
            pltpu.make_async_copy(ones_v, degi_sh.at[didx_v.at[i]],
                                  semb).wait()

        issue(0)
        issue(1)

        def body(i, carry):
            issue(i)
            drain(i - 2)
            return carry

        lax.fori_loop(2, NCH_D, body, 0)
        drain(NCH_D - 2)
        drain(NCH_D - 1)
        plsc.subcore_barrier()

        @pl.when(s == 0)
        def _writeout():
            pltpu.sync_copy(dego_sh, out_h.at[c, 0])
            pltpu.sync_copy(degi_sh, out_h.at[c, 1])

    NB = 4

    @functools.partial(
        pl.kernel,
        out_type=jax.ShapeDtypeStruct((NC, NS, ROWS_PT, D), jnp.float32),
        mesh=mesh,
        scratch_types=(
            [pltpu.VMEM((CH_A,), jnp.int32) for _ in range(2 * NB)]
            + [pltpu.VMEM((CH_A, D), jnp.float32) for _ in range(NB)]
            + [pltpu.VMEM_SHARED((N_NODES, D), jnp.float32)]
            + [pltpu.SemaphoreType.DMA for _ in range(4 * NB)]
        ),
    )
    def agg_kernel(m_h, src_h, dst_h, z2d_h, out_h, *scr):
        c = lax.axis_index("c")
        s = lax.axis_index("s")
        wid = c * NS + s
        base = wid * EPT
        sidx = scr[0:NB]
        didx = scr[NB:2 * NB]
        rows = scr[2 * NB:3 * NB]
        acc_sh = scr[3 * NB]
        semis = scr[3 * NB + 1:3 * NB + 1 + NB]
        semid = scr[3 * NB + 1 + NB:3 * NB + 1 + 2 * NB]
        semg = scr[3 * NB + 1 + 2 * NB:3 * NB + 1 + 3 * NB]
        sems = scr[3 * NB + 1 + 3 * NB:3 * NB + 1 + 4 * NB]

        def i_issue(i, b):
            off = base + i * CH_A
            pltpu.async_copy(src_h.at[pl.ds(off, CH_A)], sidx[b], semis[b])
            pltpu.async_copy(dst_h.at[pl.ds(off, CH_A)], didx[b], semid[b])

        def is_wait(i, b):
            off = base + i * CH_A
            pltpu.make_async_copy(src_h.at[pl.ds(off, CH_A)], sidx[b],
                                  semis[b]).wait()

        def id_wait(i, b):
            off = base + i * CH_A
            pltpu.make_async_copy(dst_h.at[pl.ds(off, CH_A)], didx[b],
                                  semid[b]).wait()

        def g_issue(i, b):
            pltpu.async_copy(m_h.at[sidx[b]], rows[b], semg[b])

        def g_wait(i, b):
            pltpu.make_async_copy(m_h.at[sidx[b]], rows[b], semg[b]).wait()

        def s_issue(i, b):
            pltpu.async_copy(rows[b], acc_sh.at[didx[b]], sems[b],
                             add=True)

        def s_wait(i, b):
            pltpu.make_async_copy(rows[b], acc_sh.at[didx[b]],
                                  sems[b]).wait()

        for b in range(NB):
            i_issue(b, b)
        is_wait(0, 0)
        g_issue(0, 0)
        pltpu.sync_copy(z2d_h, acc_sh.at[pl.ds(s * ROWS_PT, ROWS_PT)])
        plsc.subcore_barrier()

        def visit(i, b, g=None):
            g_wait(i, b)
            id_wait(i, b)
            s_issue(i, b)
            b2 = (b + 2) % NB
            if g is None:
                s_wait(i - 2, b2)
                i_issue(i + 2, b2)
            else:
                lo, hi = g
                if lo is not None or hi is not None:
                    conds = []
                    if lo is not None:
                        conds.append(lo)
                    if hi is not None:
                        conds.append(hi)
                    cond = conds[0] if len(conds) == 1 else (
                        jnp.logical_and(conds[0], conds[1]))

                    @pl.when(cond)
                    def _():
                        s_wait(i - 2, b2)
                        i_issue(i + 2, b2)
            b1 = (b + 1) % NB
            is_wait(i + 1, b1)
            g_issue(i + 1, b1)

        def body(g, carry):
            for b in range(NB):
                i = g * NB + b
                if b < 2:
                    visit(i, b, g=(g >= 1, None))
                elif b == 2:
                    visit(i, b)
                else:
                    visit(i, b, g=(None, g <= (NCH_A - 6) // NB))
            return carry

        lax.fori_loop(0, (NCH_A - 1) // NB, body, 0)
        il = NCH_A - 1
        bl = il % NB
        g_wait(il, bl)
        id_wait(il, bl)
        s_issue(il, bl)
        s_wait(il - 2, (bl + 2) % NB)
        s_wait(il - 1, (bl + 3) % NB)
        s_wait(il, bl)
        plsc.subcore_barrier()

        pltpu.sync_copy(acc_sh.at[pl.ds(s * ROWS_PT, ROWS_PT)],
                        out_h.at[c, s])

    return deg_kernel, agg_kernel


_RB = 1000
_NRB = N_NODES // _RB


def _prep_body(h_ref, degt_ref, m_ref, ns_ref, nd_ref):
    d = degt_ref[...]
    deg_out = d[:, 0:1] + d[:, 1:2]
    deg_in = d[:, 2:3] + d[:, 3:4]
    ns = lax.rsqrt(jnp.maximum(deg_out, 1.0))
    nd = lax.rsqrt(jnp.maximum(deg_in, 1.0))
    ns_ref[...] = ns
    nd_ref[...] = nd
    m_ref[...] = h_ref[...] * ns


_prep_call = pl.pallas_call(
    _prep_body,
    grid=(_NRB,),
    in_specs=[
        pl.BlockSpec((_RB, D), lambda i: (i, 0)),
        pl.BlockSpec((_RB, 4), lambda i: (i, 0)),
    ],
    out_specs=[
        pl.BlockSpec((_RB, D), lambda i: (i, 0)),
        pl.BlockSpec((_RB, 1), lambda i: (i, 0)),
        pl.BlockSpec((_RB, 1), lambda i: (i, 0)),
    ],
    out_shape=[
        jax.ShapeDtypeStruct((N_NODES, D), jnp.float32),
        jax.ShapeDtypeStruct((N_NODES, 1), jnp.float32),
        jax.ShapeDtypeStruct((N_NODES, 1), jnp.float32),
    ],
)


def _silu(y):
    return y * (1.0 / (1.0 + jnp.exp(-y)))


def _layer_body(p_ref, nd_ref, ns_ref, w_ref, b_ref, m_ref):
    p = p_ref[...]
    agg = (p[0] + p[1]) * nd_ref[...]
    y = jnp.dot(agg, w_ref[...], preferred_element_type=jnp.float32)
    y = y + b_ref[...]
    m_ref[...] = _silu(y) * ns_ref[...]


_layer_call = pl.pallas_call(
    _layer_body,
    grid=(_NRB,),
    in_specs=[
        pl.BlockSpec((NC, _RB, D), lambda i: (0, i, 0)),
        pl.BlockSpec((_RB, 1), lambda i: (i, 0)),
        pl.BlockSpec((_RB, 1), lambda i: (i, 0)),
        pl.BlockSpec((D, D), lambda i: (0, 0)),
        pl.BlockSpec((1, D), lambda i: (0, 0)),
    ],
    out_specs=pl.BlockSpec((_RB, D), lambda i: (i, 0)),
    out_shape=jax.ShapeDtypeStruct((N_NODES, D), jnp.float32),
)


def _final_body(p_ref, nd_ref, w_ref, b_ref, out_ref):
    i = pl.program_id(0)
    p = p_ref[...]
    agg = (p[0] + p[1]) * nd_ref[...]
    y = jnp.dot(agg, w_ref[...], preferred_element_type=jnp.float32)
    y = _silu(y + b_ref[...])
    part = jnp.sum(y, axis=0, keepdims=True) * (1.0 / N_NODES)

    @pl.when(i == 0)
    def _init():
        out_ref[...] = part

    @pl.when(i > 0)
    def _acc():
        out_ref[...] += part


_final_call = pl.pallas_call(
    _final_body,
    grid=(_NRB,),
    in_specs=[
        pl.BlockSpec((NC, _RB, D), lambda i: (0, i, 0)),
        pl.BlockSpec((_RB, 1), lambda i: (i, 0)),
        pl.BlockSpec((D, D), lambda i: (0, 0)),
        pl.BlockSpec((1, D), lambda i: (0, 0)),
    ],
    out_specs=pl.BlockSpec((1, D), lambda i: (0, 0)),
    out_shape=jax.ShapeDtypeStruct((1, D), jnp.float32),
)


def kernel(h, edge_index, W1, b1, W2, b2, W3, b3):
    src = edge_index[0].astype(jnp.int32)
    dst = edge_index[1].astype(jnp.int32)
    src_d = src.reshape(NW, NCH_D, CH_D)
    dst_d = dst.reshape(NW, NCH_D, CH_D)

    z1d = jnp.zeros((N_NODES,), jnp.float32)
    z2d = jnp.zeros((ROWS_PT, D), jnp.float32)
    ones = jnp.ones((CH_D,), jnp.float32)

    _deg_kernel, _agg_kernel = _sc_kernels()
    deg = _deg_kernel(src_d, dst_d, z1d, ones)
    degt = jnp.transpose(deg.reshape(NC * 2, N_NODES))
    degt = degt[:, jnp.array([0, 2, 1, 3])]

    m, ns, nd = _prep_call(h, degt)

    for (W, b) in ((W1, b1), (W2, b2)):
        p = _agg_kernel(m, src, dst, z2d)
        p = p.reshape(NC, N_NODES, D)
        m = _layer_call(p, nd, ns, W, b.reshape(1, D))

    p = _agg_kernel(m, src, dst, z2d)
    p = p.reshape(NC, N_NODES, D)
    return _final_call(p, nd, W3, b3.reshape(1, D))


# speedup vs baseline: 1.0035x; 1.0035x over previous
import functools

import jax
import jax.numpy as jnp
from jax import lax
from jax.experimental import pallas as pl
from jax.experimental.pallas import tpu as pltpu
from jax.experimental.pallas import tpu_sc as plsc

N_NODES = 10000
N_EDGES = 320000
D = 128

NC = 2
NS = 16
NW = NC * NS
EPT = N_EDGES // NW

CH_D = 125
NCH_D = EPT // CH_D

CH_A = 80
NCH_A = EPT // CH_A

ROWS_PT = N_NODES // NS


@functools.cache
def _sc_kernels():
    mesh = plsc.VectorSubcoreMesh(
        core_axis_name="c", subcore_axis_name="s", num_cores=NC,
        num_subcores=NS
    )

    @functools.partial(
        pl.kernel,
        out_type=jax.ShapeDtypeStruct((NC, 2, N_NODES), jnp.float32),
        mesh=mesh,
        scratch_types=[
            pltpu.VMEM((NCH_D, CH_D), jnp.int32),
            pltpu.VMEM((NCH_D, CH_D), jnp.int32),
            pltpu.VMEM((CH_D,), jnp.float32),
            pltpu.VMEM_SHARED((N_NODES,), jnp.float32),
            pltpu.VMEM_SHARED((N_NODES,), jnp.float32),
            pltpu.SemaphoreType.DMA,
            pltpu.SemaphoreType.DMA,
        ],
    )
    def deg_kernel(src_h, dst_h, z1d_h, ones_h, out_h, sidx_v, didx_v,
                   ones_v, dego_sh, degi_sh, sema, semb):
        c = lax.axis_index("c")
        s = lax.axis_index("s")
        wid = c * NS + s

        @pl.when(s == 0)
        def _zero():
            pltpu.sync_copy(z1d_h, dego_sh)
            pltpu.sync_copy(z1d_h, degi_sh)

        pltpu.sync_copy(src_h.at[wid], sidx_v)
        pltpu.sync_copy(dst_h.at[wid], didx_v)
        pltpu.sync_copy(ones_h, ones_v)
        plsc.subcore_barrier()

        def issue(i):
            pltpu.async_copy(ones_v, dego_sh.at[sidx_v.at[i]], sema,
                             add=True)
            pltpu.async_copy(ones_v, degi_sh.at[didx_v.at[i]], semb,
                             add=True)

        def drain(i):
            pltpu.make_async_copy(ones_v, dego_sh.at[sidx_v.at[i]],
                                  sema).wait()
            pltpu.make_async_copy(ones_v, degi_sh.at[didx_v.at[i]],
                                  semb).wait()

        issue(0)
        issue(1)

        def body(i, carry):
            issue(i)
            drain(i - 2)
            return carry

        lax.fori_loop(2, NCH_D, body, 0)
        drain(NCH_D - 2)
        drain(NCH_D - 1)
        plsc.subcore_barrier()

        @pl.when(s == 0)
        def _writeout():
            pltpu.sync_copy(dego_sh, out_h.at[c, 0])
            pltpu.sync_copy(degi_sh, out_h.at[c, 1])

    NB = 4

    @functools.partial(
        pl.kernel,
        out_type=jax.ShapeDtypeStruct((NC, NS, ROWS_PT, D), jnp.float32),
        mesh=mesh,
        scratch_types=(
            [pltpu.VMEM((CH_A,), jnp.int32) for _ in range(2 * NB)]
            + [pltpu.VMEM((CH_A, D), jnp.float32) for _ in range(NB)]
            + [pltpu.VMEM_SHARED((N_NODES, D), jnp.float32)]
            + [pltpu.SemaphoreType.DMA for _ in range(4 * NB)]
        ),
    )
    def agg_kernel(m_h, src_h, dst_h, z2d_h, out_h, *scr):
        c = lax.axis_index("c")
        s = lax.axis_index("s")
        wid = c * NS + s
        base = wid * EPT
        sidx = scr[0:NB]
        didx = scr[NB:2 * NB]
        rows = scr[2 * NB:3 * NB]
        acc_sh = scr[3 * NB]
        semis = scr[3 * NB + 1:3 * NB + 1 + NB]
        semid = scr[3 * NB + 1 + NB:3 * NB + 1 + 2 * NB]
        semg = scr[3 * NB + 1 + 2 * NB:3 * NB + 1 + 3 * NB]
        sems = scr[3 * NB + 1 + 3 * NB:3 * NB + 1 + 4 * NB]

        def i_issue(i, b):
            off = base + i * CH_A
            pltpu.async_copy(src_h.at[pl.ds(off, CH_A)], sidx[b], semis[b])
            pltpu.async_copy(dst_h.at[pl.ds(off, CH_A)], didx[b], semid[b])

        def is_wait(i, b):
            off = base + i * CH_A
            pltpu.make_async_copy(src_h.at[pl.ds(off, CH_A)], sidx[b],
                                  semis[b]).wait()

        def id_wait(i, b):
            off = base + i * CH_A
            pltpu.make_async_copy(dst_h.at[pl.ds(off, CH_A)], didx[b],
                                  semid[b]).wait()

        def g_issue(i, b):
            pltpu.async_copy(m_h.at[sidx[b]], rows[b], semg[b])

        def g_wait(i, b):
            pltpu.make_async_copy(m_h.at[sidx[b]], rows[b], semg[b]).wait()

        def s_issue(i, b):
            pltpu.async_copy(rows[b], acc_sh.at[didx[b]], sems[b],
                             add=True)

        def s_wait(i, b):
            pltpu.make_async_copy(rows[b], acc_sh.at[didx[b]],
                                  sems[b]).wait()

        for b in range(NB):
            i_issue(b, b)
        is_wait(0, 0)
        g_issue(0, 0)
        pltpu.sync_copy(z2d_h, acc_sh.at[pl.ds(s * ROWS_PT, ROWS_PT)])
        plsc.subcore_barrier()

        def visit(i, b, g=None):
            g_wait(i, b)
            id_wait(i, b)
            s_issue(i, b)
            b2 = (b + 2) % NB
            if g is None:
                s_wait(i - 2, b2)
                i_issue(i + 2, b2)
            else:
                lo, hi = g
                if lo is not None or hi is not None:
                    conds = []
                    if lo is not None:
                        conds.append(lo)
                    if hi is not None:
                        conds.append(hi)
                    cond = conds[0] if len(conds) == 1 else (
                        jnp.logical_and(conds[0], conds[1]))

                    @pl.when(cond)
                    def _():
                        s_wait(i - 2, b2)
                        i_issue(i + 2, b2)
            b1 = (b + 1) % NB
            is_wait(i + 1, b1)
            g_issue(i + 1, b1)

        def body(g, carry):
            for b in range(NB):
                i = g * NB + b
                if b < 2:
                    visit(i, b, g=(g >= 1, None))
                elif b == 2:
                    visit(i, b)
                else:
                    visit(i, b, g=(None, g <= (NCH_A - 6) // NB))
            return carry

        lax.fori_loop(0, (NCH_A - 1) // NB, body, 0)
        il = NCH_A - 1
        bl = il % NB
        g_wait(il, bl)
        id_wait(il, bl)
        s_issue(il, bl)
        s_wait(il - 2, (bl + 2) % NB)
        s_wait(il - 1, (bl + 3) % NB)
        s_wait(il, bl)
        plsc.subcore_barrier()

        pltpu.sync_copy(acc_sh.at[pl.ds(s * ROWS_PT, ROWS_PT)],
                        out_h.at[c, s])

    return deg_kernel, agg_kernel


_RB = 1000
_NRB = N_NODES // _RB


def _prep_body(h_ref, degt_ref, m_ref, ns_ref, nd_ref):
    d = degt_ref[...]
    deg_out = d[:, 0:1] + d[:, 2:3]
    deg_in = d[:, 1:2] + d[:, 3:4]
    ns = lax.rsqrt(jnp.maximum(deg_out, 1.0))
    nd = lax.rsqrt(jnp.maximum(deg_in, 1.0))
    ns_ref[...] = ns
    nd_ref[...] = nd
    m_ref[...] = h_ref[...] * ns


_prep_call = pl.pallas_call(
    _prep_body,
    grid=(_NRB,),
    in_specs=[
        pl.BlockSpec((_RB, D), lambda i: (i, 0)),
        pl.BlockSpec((_RB, 4), lambda i: (i, 0)),
    ],
    out_specs=[
        pl.BlockSpec((_RB, D), lambda i: (i, 0)),
        pl.BlockSpec((_RB, 1), lambda i: (i, 0)),
        pl.BlockSpec((_RB, 1), lambda i: (i, 0)),
    ],
    out_shape=[
        jax.ShapeDtypeStruct((N_NODES, D), jnp.float32),
        jax.ShapeDtypeStruct((N_NODES, 1), jnp.float32),
        jax.ShapeDtypeStruct((N_NODES, 1), jnp.float32),
    ],
)


def _silu(y):
    return y * (1.0 / (1.0 + jnp.exp(-y)))


def _layer_body(p_ref, nd_ref, ns_ref, w_ref, b_ref, m_ref):
    p = p_ref[...]
    agg = (p[0] + p[1]) * nd_ref[...]
    y = jnp.dot(agg, w_ref[...], preferred_element_type=jnp.float32)
    y = y + b_ref[...]
    m_ref[...] = _silu(y) * ns_ref[...]


_layer_call = pl.pallas_call(
    _layer_body,
    grid=(_NRB,),
    in_specs=[
        pl.BlockSpec((NC, _RB, D), lambda i: (0, i, 0)),
        pl.BlockSpec((_RB, 1), lambda i: (i, 0)),
        pl.BlockSpec((_RB, 1), lambda i: (i, 0)),
        pl.BlockSpec((D, D), lambda i: (0, 0)),
        pl.BlockSpec((1, D), lambda i: (0, 0)),
    ],
    out_specs=pl.BlockSpec((_RB, D), lambda i: (i, 0)),
    out_shape=jax.ShapeDtypeStruct((N_NODES, D), jnp.float32),
)


def _final_body(p_ref, nd_ref, w_ref, b_ref, out_ref):
    i = pl.program_id(0)
    p = p_ref[...]
    agg = (p[0] + p[1]) * nd_ref[...]
    y = jnp.dot(agg, w_ref[...], preferred_element_type=jnp.float32)
    y = _silu(y + b_ref[...])
    part = jnp.sum(y, axis=0, keepdims=True) * (1.0 / N_NODES)

    @pl.when(i == 0)
    def _init():
        out_ref[...] = part

    @pl.when(i > 0)
    def _acc():
        out_ref[...] += part


_final_call = pl.pallas_call(
    _final_body,
    grid=(_NRB,),
    in_specs=[
        pl.BlockSpec((NC, _RB, D), lambda i: (0, i, 0)),
        pl.BlockSpec((_RB, 1), lambda i: (i, 0)),
        pl.BlockSpec((D, D), lambda i: (0, 0)),
        pl.BlockSpec((1, D), lambda i: (0, 0)),
    ],
    out_specs=pl.BlockSpec((1, D), lambda i: (0, 0)),
    out_shape=jax.ShapeDtypeStruct((1, D), jnp.float32),
)


def kernel(h, edge_index, W1, b1, W2, b2, W3, b3):
    src = edge_index[0].astype(jnp.int32)
    dst = edge_index[1].astype(jnp.int32)
    src_d = src.reshape(NW, NCH_D, CH_D)
    dst_d = dst.reshape(NW, NCH_D, CH_D)

    z1d = jnp.zeros((N_NODES,), jnp.float32)
    z2d = jnp.zeros((ROWS_PT, D), jnp.float32)
    ones = jnp.ones((CH_D,), jnp.float32)

    _deg_kernel, _agg_kernel = _sc_kernels()
    deg = _deg_kernel(src_d, dst_d, z1d, ones)
    degt = jnp.transpose(deg.reshape(NC * 2, N_NODES))

    m, ns, nd = _prep_call(h, degt)

    for (W, b) in ((W1, b1), (W2, b2)):
        p = _agg_kernel(m, src, dst, z2d)
        p = p.reshape(NC, N_NODES, D)
        m = _layer_call(p, nd, ns, W, b.reshape(1, D))

    p = _agg_kernel(m, src, dst, z2d)
    p = p.reshape(NC, N_NODES, D)
    return _final_call(p, nd, W3, b3.reshape(1, D))


# speedup vs baseline: 1.1605x; 1.1565x over previous
import functools

import jax
import jax.numpy as jnp
from jax import lax
from jax.experimental import pallas as pl
from jax.experimental.pallas import tpu as pltpu
from jax.experimental.pallas import tpu_sc as plsc

N_NODES = 10000
N_EDGES = 320000
D = 128

NC = 2
NS = 16
NW = NC * NS
EPT = N_EDGES // NW

CH_D = 125
NCH_D = EPT // CH_D

CH_A = 125
NCH_A = EPT // CH_A

ROWS_PT = N_NODES // NS


@functools.cache
def _sc_kernels():
    mesh = plsc.VectorSubcoreMesh(
        core_axis_name="c", subcore_axis_name="s", num_cores=NC,
        num_subcores=NS
    )

    @functools.partial(
        pl.kernel,
        out_type=jax.ShapeDtypeStruct((NC, 2, N_NODES), jnp.float32),
        mesh=mesh,
        scratch_types=[
            pltpu.VMEM((NCH_D, CH_D), jnp.int32),
            pltpu.VMEM((NCH_D, CH_D), jnp.int32),
            pltpu.VMEM((CH_D,), jnp.float32),
            pltpu.VMEM_SHARED((N_NODES,), jnp.float32),
            pltpu.VMEM_SHARED((N_NODES,), jnp.float32),
            pltpu.SemaphoreType.DMA,
            pltpu.SemaphoreType.DMA,
        ],
    )
    def deg_kernel(src_h, dst_h, z1d_h, ones_h, out_h, sidx_v, didx_v,
                   ones_v, dego_sh, degi_sh, sema, semb):
        c = lax.axis_index("c")
        s = lax.axis_index("s")
        wid = c * NS + s

        @pl.when(s == 0)
        def _zero():
            pltpu.sync_copy(z1d_h, dego_sh)
            pltpu.sync_copy(z1d_h, degi_sh)

        pltpu.sync_copy(src_h.at[wid], sidx_v)
        pltpu.sync_copy(dst_h.at[wid], didx_v)
        pltpu.sync_copy(ones_h, ones_v)
        plsc.subcore_barrier()

        def issue(i):
            pltpu.async_copy(ones_v, dego_sh.at[sidx_v.at[i]], sema,
                             add=True)
            pltpu.async_copy(ones_v, degi_sh.at[didx_v.at[i]], semb,
                             add=True)

        def drain(i):
            pltpu.make_async_copy(ones_v, dego_sh.at[sidx_v.at[i]],
                                  sema).wait()
            pltpu.make_async_copy(ones_v, degi_sh.at[didx_v.at[i]],
                                  semb).wait()

        issue(0)
        issue(1)

        def body(i, carry):
            issue(i)
            drain(i - 2)
            return carry

        lax.fori_loop(2, NCH_D, body, 0)
        drain(NCH_D - 2)
        drain(NCH_D - 1)
        plsc.subcore_barrier()

        @pl.when(s == 0)
        def _writeout():
            pltpu.sync_copy(dego_sh, out_h.at[c, 0])
            pltpu.sync_copy(degi_sh, out_h.at[c, 1])

    @functools.partial(
        pl.kernel,
        out_type=jax.ShapeDtypeStruct((NC, NS, ROWS_PT, D), jnp.float32),
        mesh=mesh,
        scratch_types=[
            pltpu.VMEM((NCH_A, CH_A), jnp.int32),
            pltpu.VMEM((CH_A,), jnp.int32),
            pltpu.VMEM((CH_A,), jnp.int32),
            pltpu.VMEM((CH_A, D), jnp.float32),
            pltpu.VMEM((CH_A, D), jnp.float32),
            pltpu.VMEM_SHARED((N_NODES, D), jnp.float32),
            pltpu.SemaphoreType.DMA,
            pltpu.SemaphoreType.DMA,
            pltpu.SemaphoreType.DMA,
            pltpu.SemaphoreType.DMA,
            pltpu.SemaphoreType.DMA,
            pltpu.SemaphoreType.DMA,
        ],
    )
    def agg_kernel(m_h, src_h, dst_h, z2d_h, out_h, sidx_v, didx0, didx1,
                   rows0, rows1, acc_sh, sg0, sg1, ss0, ss1, sd0, sd1):
        c = lax.axis_index("c")
        s = lax.axis_index("s")
        wid = c * NS + s
        rows = (rows0, rows1)
        didx = (didx0, didx1)
        semg = (sg0, sg1)
        sems = (ss0, ss1)
        semd = (sd0, sd1)

        def g_issue(i, b):
            pltpu.async_copy(m_h.at[sidx_v.at[i]], rows[b], semg[b])

        def g_wait(i, b):
            pltpu.make_async_copy(m_h.at[sidx_v.at[i]], rows[b],
                                  semg[b]).wait()

        def s_issue(i, b):
            pltpu.async_copy(rows[b], acc_sh.at[didx[b]], sems[b],
                             add=True)

        def s_wait(i, b):
            pltpu.make_async_copy(rows[b], acc_sh.at[didx[b]],
                                  sems[b]).wait()

        def d_issue(i, b):
            pltpu.async_copy(dst_h.at[wid, i], didx[b], semd[b])

        def d_wait(i, b):
            pltpu.make_async_copy(dst_h.at[wid, i], didx[b], semd[b]).wait()

        pltpu.sync_copy(src_h.at[wid], sidx_v)
        d_issue(0, 0)
        g_issue(0, 0)
        d_issue(1, 1)
        pltpu.sync_copy(z2d_h, acc_sh.at[pl.ds(s * ROWS_PT, ROWS_PT)])
        plsc.subcore_barrier()

        g_wait(0, 0)
        d_wait(0, 0)
        s_issue(0, 0)
        g_issue(1, 1)

        def visit(i, b):
            g_wait(i, b)
            d_wait(i, b)
            s_issue(i, b)
            s_wait(i - 1, 1 - b)
            d_issue(i + 1, 1 - b)
            g_issue(i + 1, 1 - b)

        def body(g, carry):
            visit(2 * g + 1, 1)
            visit(2 * g + 2, 0)
            return carry

        lax.fori_loop(0, (NCH_A - 3) // 2, body, 0)
        visit(NCH_A - 3, (NCH_A - 3) % 2)
        visit(NCH_A - 2, (NCH_A - 2) % 2)
        bl = (NCH_A - 1) % 2
        g_wait(NCH_A - 1, bl)
        d_wait(NCH_A - 1, bl)
        s_issue(NCH_A - 1, bl)
        s_wait(NCH_A - 2, 1 - bl)
        s_wait(NCH_A - 1, bl)
        plsc.subcore_barrier()

        pltpu.sync_copy(acc_sh.at[pl.ds(s * ROWS_PT, ROWS_PT)],
                        out_h.at[c, s])

    return deg_kernel, agg_kernel


_RB = 1000
_NRB = N_NODES // _RB


def _prep_body(h_ref, degt_ref, m_ref, ns_ref, nd_ref):
    d = degt_ref[...]
    deg_out = d[:, 0:1] + d[:, 2:3]
    deg_in = d[:, 1:2] + d[:, 3:4]
    ns = lax.rsqrt(jnp.maximum(deg_out, 1.0))
    nd = lax.rsqrt(jnp.maximum(deg_in, 1.0))
    ns_ref[...] = ns
    nd_ref[...] = nd
    m_ref[...] = h_ref[...] * ns


_prep_call = pl.pallas_call(
    _prep_body,
    grid=(_NRB,),
    in_specs=[
        pl.BlockSpec((_RB, D), lambda i: (i, 0)),
        pl.BlockSpec((_RB, 4), lambda i: (i, 0)),
    ],
    out_specs=[
        pl.BlockSpec((_RB, D), lambda i: (i, 0)),
        pl.BlockSpec((_RB, 1), lambda i: (i, 0)),
        pl.BlockSpec((_RB, 1), lambda i: (i, 0)),
    ],
    out_shape=[
        jax.ShapeDtypeStruct((N_NODES, D), jnp.float32),
        jax.ShapeDtypeStruct((N_NODES, 1), jnp.float32),
        jax.ShapeDtypeStruct((N_NODES, 1), jnp.float32),
    ],
)


def _silu(y):
    return y * (1.0 / (1.0 + jnp.exp(-y)))


def _layer_body(p_ref, nd_ref, ns_ref, w_ref, b_ref, m_ref):
    p = p_ref[...]
    agg = (p[0] + p[1]) * nd_ref[...]
    y = jnp.dot(agg, w_ref[...], preferred_element_type=jnp.float32)
    y = y + b_ref[...]
    m_ref[...] = _silu(y) * ns_ref[...]


_layer_call = pl.pallas_call(
    _layer_body,
    grid=(_NRB,),
    in_specs=[
        pl.BlockSpec((NC, _RB, D), lambda i: (0, i, 0)),
        pl.BlockSpec((_RB, 1), lambda i: (i, 0)),
        pl.BlockSpec((_RB, 1), lambda i: (i, 0)),
        pl.BlockSpec((D, D), lambda i: (0, 0)),
        pl.BlockSpec((1, D), lambda i: (0, 0)),
    ],
    out_specs=pl.BlockSpec((_RB, D), lambda i: (i, 0)),
    out_shape=jax.ShapeDtypeStruct((N_NODES, D), jnp.float32),
)


def _final_body(p_ref, nd_ref, w_ref, b_ref, out_ref):
    i = pl.program_id(0)
    p = p_ref[...]
    agg = (p[0] + p[1]) * nd_ref[...]
    y = jnp.dot(agg, w_ref[...], preferred_element_type=jnp.float32)
    y = _silu(y + b_ref[...])
    part = jnp.sum(y, axis=0, keepdims=True) * (1.0 / N_NODES)

    @pl.when(i == 0)
    def _init():
        out_ref[...] = part

    @pl.when(i > 0)
    def _acc():
        out_ref[...] += part


_final_call = pl.pallas_call(
    _final_body,
    grid=(_NRB,),
    in_specs=[
        pl.BlockSpec((NC, _RB, D), lambda i: (0, i, 0)),
        pl.BlockSpec((_RB, 1), lambda i: (i, 0)),
        pl.BlockSpec((D, D), lambda i: (0, 0)),
        pl.BlockSpec((1, D), lambda i: (0, 0)),
    ],
    out_specs=pl.BlockSpec((1, D), lambda i: (0, 0)),
    out_shape=jax.ShapeDtypeStruct((1, D), jnp.float32),
)


def kernel(h, edge_index, W1, b1, W2, b2, W3, b3):
    src = edge_index[0].astype(jnp.int32)
    dst = edge_index[1].astype(jnp.int32)
    src_d = src.reshape(NW, NCH_D, CH_D)
    dst_d = dst.reshape(NW, NCH_D, CH_D)

    src_a = src.reshape(NW, NCH_A, CH_A)
    dst_a = dst.reshape(NW, NCH_A, CH_A)
    z1d = jnp.zeros((N_NODES,), jnp.float32)
    z2d = jnp.zeros((ROWS_PT, D), jnp.float32)
    ones = jnp.ones((CH_D,), jnp.float32)

    _deg_kernel, _agg_kernel = _sc_kernels()
    deg = _deg_kernel(src_d, dst_d, z1d, ones)
    degt = jnp.transpose(deg.reshape(NC * 2, N_NODES))

    m, ns, nd = _prep_call(h, degt)

    for (W, b) in ((W1, b1), (W2, b2)):
        p = _agg_kernel(m, src_a, dst_a, z2d)
        p = p.reshape(NC, N_NODES, D)
        m = _layer_call(p, nd, ns, W, b.reshape(1, D))

    p = _agg_kernel(m, src_a, dst_a, z2d)
    p = p.reshape(NC, N_NODES, D)
    return _final_call(p, nd, W3, b3.reshape(1, D))
